# baseline (speedup 1.0000x reference)
import functools

import jax
import jax.numpy as jnp
from jax import lax
from jax.experimental import pallas as pl
from jax.experimental.pallas import tpu as pltpu
from jax.experimental.pallas import tpu_sc as plsc

E = 16
TOPK = 2
DIM = 2048
INTER = 1024
T = 2048
TB = 128
G = (T * TOPK) // TB + E
NROWS = G * TB

NC = 2
NS = 16
L = 16
NW = NC * NS

TPW = T // NW
DCH = 16
DNCH = TPW // DCH
CCH = 8


def _router_body(x_ref, gw_ref, topv_ref, topi_ref):
    xb = x_ref[...]
    gw = gw_ref[...]
    logits = lax.dot_general(xb, gw, (((1,), (1,)), ((), ())),
                             preferred_element_type=jnp.float32)
    m = jnp.max(logits, axis=1, keepdims=True)
    ex = jnp.exp(logits - m)
    s = ex / jnp.sum(ex, axis=1, keepdims=True)
    col = lax.broadcasted_iota(jnp.int32, s.shape, 1)
    v1 = jnp.max(s, axis=1, keepdims=True)
    i1 = jnp.min(jnp.where(s == v1, col, E), axis=1, keepdims=True)
    s2 = jnp.where(col == i1, jnp.float32(-jnp.inf), s)
    v2 = jnp.max(s2, axis=1, keepdims=True)
    i2 = jnp.min(jnp.where(s2 == v2, col, E), axis=1, keepdims=True)
    topv_ref[...] = jnp.concatenate([v1, v2], axis=1)
    topi_ref[...] = jnp.concatenate([i1, i2], axis=1)


def _router(xf, gate_w):
    rb = 256
    return pl.pallas_call(
        _router_body,
        grid=(T // rb,),
        in_specs=[
            pl.BlockSpec((rb, DIM), lambda g: (g, 0)),
            pl.BlockSpec((E, DIM), lambda g: (0, 0)),
        ],
        out_specs=[
            pl.BlockSpec((rb, TOPK), lambda g: (g, 0)),
            pl.BlockSpec((rb, TOPK), lambda g: (g, 0)),
        ],
        out_shape=[
            jax.ShapeDtypeStruct((T, TOPK), jnp.float32),
            jax.ShapeDtypeStruct((T, TOPK), jnp.int32),
        ],
    )(xf, gate_w)


def _route_metadata(topi):
    flat_e = topi.reshape(-1)
    eids = jnp.arange(E, dtype=jnp.int32)
    oh = (flat_e[:, None] == eids[None, :]).astype(jnp.int32)
    csum = jnp.cumsum(oh, axis=0)
    rank = jnp.sum(csum * oh, axis=1) - 1
    counts = csum[-1]
    padded = ((counts + TB - 1) // TB) * TB
    ends = jnp.cumsum(padded)
    offs = ends - padded
    dest = jnp.sum(oh * offs[None, :], axis=1) + rank
    cb = ends // TB
    gids = jnp.arange(G, dtype=jnp.int32)
    block_expert = jnp.minimum(
        jnp.sum((gids[:, None] >= cb[None, :]).astype(jnp.int32), axis=1),
        E - 1).astype(jnp.int32)
    block_valid = (gids < cb[E - 1]).astype(jnp.int32)
    d2 = dest.reshape(T, TOPK)
    return d2[:, 0], d2[:, 1], block_expert, block_valid


def _sc_dispatch_body(x_hbm, d0_hbm, d1_hbm, out_hbm,
                      d0_v, d1_v, buf0, buf1, rs0, rs1, s00, s01, s10, s11):
    wid = lax.axis_index("s") * NC + lax.axis_index("c")
    base = wid * TPW
    pltpu.sync_copy(d0_hbm.at[wid], d0_v)
    pltpu.sync_copy(d1_hbm.at[wid], d1_v)
    bufs = (buf0, buf1)
    rsems = (rs0, rs1)
    ssems = ((s00, s01), (s10, s11))
    reads = [None, None]
    scats = [None, None, None, None]
    for c in range(DNCH):
        b = c % 2
        if scats[2 * b] is not None:
            scats[2 * b].wait()
            scats[2 * b + 1].wait()
        reads[b] = pltpu.async_copy(
            x_hbm.at[pl.ds(base + c * DCH, DCH)], bufs[b], rsems[b])
        reads[b].wait()
        scats[2 * b] = pltpu.async_copy(
            bufs[b], out_hbm.at[d0_v.at[c]], ssems[b][0])
        scats[2 * b + 1] = pltpu.async_copy(
            bufs[b], out_hbm.at[d1_v.at[c]], ssems[b][1])
    for cp in scats:
        if cp is not None:
            cp.wait()


def _sc_dispatch(xf, d0r, d1r):
    mesh = plsc.VectorSubcoreMesh(core_axis_name="c", subcore_axis_name="s")
    f = functools.partial(
        pl.kernel,
        mesh=mesh,
        out_type=jax.ShapeDtypeStruct((NROWS, DIM), jnp.float32),
        scratch_types=(
            [pltpu.VMEM((DNCH, DCH), jnp.int32)] * 2
            + [pltpu.VMEM((DCH, DIM), jnp.float32)] * 2
            + [pltpu.SemaphoreType.DMA] * 6
        ),
    )(_sc_dispatch_body)
    return f(xf, d0r, d1r)


def _gmm_body(be_ref, bv_ref, x_ref, w1_ref, w3_ref, w2_ref, out_ref):
    g = pl.program_id(0)

    @pl.when(bv_ref[g] != 0)
    def _():
        xb = x_ref[...]
        w1e = w1_ref[0]
        w3e = w3_ref[0]
        w2e = w2_ref[0]
        h1 = lax.dot_general(xb, w1e, (((1,), (1,)), ((), ())),
                             preferred_element_type=jnp.float32)
        h3 = lax.dot_general(xb, w3e, (((1,), (1,)), ((), ())),
                             preferred_element_type=jnp.float32)
        hh = h1 * (1.0 / (1.0 + jnp.exp(-h1))) * h3
        out_ref[...] = lax.dot_general(hh, w2e, (((1,), (1,)), ((), ())),
                                       preferred_element_type=jnp.float32)


def _gmm(block_expert, block_valid, xs, w1, w3, w2):
    grid_spec = pltpu.PrefetchScalarGridSpec(
        num_scalar_prefetch=2,
        grid=(G,),
        in_specs=[
            pl.BlockSpec((TB, DIM), lambda g, be, bv: (g, 0)),
            pl.BlockSpec((1, INTER, DIM), lambda g, be, bv: (be[g], 0, 0)),
            pl.BlockSpec((1, INTER, DIM), lambda g, be, bv: (be[g], 0, 0)),
            pl.BlockSpec((1, DIM, INTER), lambda g, be, bv: (be[g], 0, 0)),
        ],
        out_specs=pl.BlockSpec((TB, DIM), lambda g, be, bv: (g, 0)),
    )
    return pl.pallas_call(
        _gmm_body,
        grid_spec=grid_spec,
        out_shape=jax.ShapeDtypeStruct((NROWS, DIM), jnp.float32),
        compiler_params=pltpu.CompilerParams(
            dimension_semantics=("arbitrary",),
            vmem_limit_bytes=100 * 1024 * 1024),
    )(block_expert, block_valid, xs, w1, w3, w2)


def _shared_body(x_ref, sw1_ref, sw3_ref, sw2_ref, z_ref):
    xb = x_ref[...]
    h1 = lax.dot_general(xb, sw1_ref[...], (((1,), (1,)), ((), ())),
                         preferred_element_type=jnp.float32)
    h3 = lax.dot_general(xb, sw3_ref[...], (((1,), (1,)), ((), ())),
                         preferred_element_type=jnp.float32)
    hh = h1 * (1.0 / (1.0 + jnp.exp(-h1))) * h3
    z_ref[...] = lax.dot_general(hh, sw2_ref[...], (((1,), (1,)), ((), ())),
                                 preferred_element_type=jnp.float32)


def _shared(xf, sw1, sw3, sw2):
    sb = 256
    return pl.pallas_call(
        _shared_body,
        grid=(T // sb,),
        in_specs=[
            pl.BlockSpec((sb, DIM), lambda g: (g, 0)),
            pl.BlockSpec((INTER, DIM), lambda g: (0, 0)),
            pl.BlockSpec((INTER, DIM), lambda g: (0, 0)),
            pl.BlockSpec((DIM, INTER), lambda g: (0, 0)),
        ],
        out_specs=pl.BlockSpec((sb, DIM), lambda g: (g, 0)),
        out_shape=jax.ShapeDtypeStruct((T, DIM), jnp.float32),
        compiler_params=pltpu.CompilerParams(
            dimension_semantics=("arbitrary",),
            vmem_limit_bytes=100 * 1024 * 1024),
    )(xf, sw1, sw3, sw2)


def _sc_combine_body(contrib_hbm, d0_hbm, d1_hbm, z_hbm, w0b_hbm, w1b_hbm,
                     out_hbm,
                     d0_v, d1_v, w0_v, w1_v,
                     r0a, r1a, za, r0b, r1b, zb,
                     sa0, sa1, sa2, sb0, sb1, sb2, wsa, wsb):
    wid = lax.axis_index("s") * NC + lax.axis_index("c")
    base = wid * TPW
    pltpu.sync_copy(d0_hbm.at[pl.ds(base, TPW)], d0_v)
    pltpu.sync_copy(d1_hbm.at[pl.ds(base, TPW)], d1_v)
    pltpu.sync_copy(w0b_hbm.at[pl.ds(base, TPW)], w0_v)
    pltpu.sync_copy(w1b_hbm.at[pl.ds(base, TPW)], w1_v)

    def accum(c, dst, s0, s1):
        wv0 = [w0_v[c * CCH + i, :] for i in range(CCH)]
        wv1 = [w1_v[c * CCH + i, :] for i in range(CCH)]

        def add_body(j, carry2):
            sl = pl.ds(j * L, L)
            for i in range(CCH):
                dst[i, sl] = dst[i, sl] + wv0[i] * s0[i, sl] + wv1[i] * s1[i, sl]
            return carry2
        lax.fori_loop(0, DIM // L, add_body, 0)

    def pair(i, carry):
        c0 = i * 2
        c1 = c0 + 1
        ta = base + c0 * CCH
        tb = base + c1 * CCH
        ga0 = pltpu.async_copy(contrib_hbm.at[d0_v.at[pl.ds(c0 * CCH, CCH)]],
                               r0a, sa0)
        ga1 = pltpu.async_copy(contrib_hbm.at[d1_v.at[pl.ds(c0 * CCH, CCH)]],
                               r1a, sa1)
        ga2 = pltpu.async_copy(z_hbm.at[pl.ds(ta, CCH)], za, sa2)
        gb0 = pltpu.async_copy(contrib_hbm.at[d0_v.at[pl.ds(c1 * CCH, CCH)]],
                               r0b, sb0)
        gb1 = pltpu.async_copy(contrib_hbm.at[d1_v.at[pl.ds(c1 * CCH, CCH)]],
                               r1b, sb1)
        gb2 = pltpu.async_copy(z_hbm.at[pl.ds(tb, CCH)], zb, sb2)
        ga0.wait()
        ga1.wait()
        ga2.wait()
        accum(c0, za, r0a, r1a)
        wa = pltpu.async_copy(za, out_hbm.at[pl.ds(ta, CCH)], wsa)
        gb0.wait()
        gb1.wait()
        gb2.wait()
        accum(c1, zb, r0b, r1b)
        wb = pltpu.async_copy(zb, out_hbm.at[pl.ds(tb, CCH)], wsb)
        wa.wait()
        wb.wait()
        return carry

    lax.fori_loop(0, TPW // (2 * CCH), pair, 0)


def _sc_combine(contrib, d0, d1, z, w0b, w1b):
    mesh = plsc.VectorSubcoreMesh(core_axis_name="c", subcore_axis_name="s")
    f = functools.partial(
        pl.kernel,
        mesh=mesh,
        out_type=jax.ShapeDtypeStruct((T, DIM), jnp.float32),
        scratch_types=(
            [pltpu.VMEM((TPW,), jnp.int32)] * 2
            + [pltpu.VMEM((TPW, L), jnp.float32)] * 2
            + [pltpu.VMEM((CCH, DIM), jnp.float32)] * 6
            + [pltpu.SemaphoreType.DMA] * 8
        ),
    )(_sc_combine_body)
    return f(contrib, d0, d1, z, w0b, w1b)


def kernel(x, gate_w, w1, w2, w3, sw1, sw2, sw3):
    shape = x.shape
    xf = x.reshape(T, DIM)
    topv, topi = _router(xf, gate_w)
    d0, d1, block_expert, block_valid = _route_metadata(topi)
    d0r = d0.reshape(NW, DNCH, DCH)
    d1r = d1.reshape(NW, DNCH, DCH)
    w0b = jnp.broadcast_to(topv[:, 0:1], (T, L))
    w1b = jnp.broadcast_to(topv[:, 1:2], (T, L))
    xs = _sc_dispatch(xf, d0r, d1r)
    z = _shared(xf, sw1, sw3, sw2)
    contrib = _gmm(block_expert, block_valid, xs, w1, w3, w2)
    y = _sc_combine(contrib, d0, d1, z, w0b, w1b)
    return y.reshape(shape)


# speedup vs baseline: 1.2470x; 1.2470x over previous
import functools

import jax
import jax.numpy as jnp
from jax import lax
from jax.experimental import pallas as pl
from jax.experimental.pallas import tpu as pltpu
from jax.experimental.pallas import tpu_sc as plsc

E = 16
TOPK = 2
DIM = 2048
INTER = 1024
T = 2048
TB = 384
G = (T * TOPK) // TB + E
NROWS = G * TB

NC = 2
NS = 16
L = 16
NW = NC * NS

TPW = T // NW
DCH = 16
DNCH = TPW // DCH
CCH = 8


def _router_body(x_ref, gw_ref, topv_ref, topi_ref):
    xb = x_ref[...]
    gw = gw_ref[...]
    logits = lax.dot_general(xb, gw, (((1,), (1,)), ((), ())),
                             preferred_element_type=jnp.float32)
    m = jnp.max(logits, axis=1, keepdims=True)
    ex = jnp.exp(logits - m)
    s = ex / jnp.sum(ex, axis=1, keepdims=True)
    col = lax.broadcasted_iota(jnp.int32, s.shape, 1)
    v1 = jnp.max(s, axis=1, keepdims=True)
    i1 = jnp.min(jnp.where(s == v1, col, E), axis=1, keepdims=True)
    s2 = jnp.where(col == i1, jnp.float32(-jnp.inf), s)
    v2 = jnp.max(s2, axis=1, keepdims=True)
    i2 = jnp.min(jnp.where(s2 == v2, col, E), axis=1, keepdims=True)
    topv_ref[...] = jnp.concatenate([v1, v2], axis=1)
    topi_ref[...] = jnp.concatenate([i1, i2], axis=1)


def _router(xf, gate_w):
    rb = 256
    return pl.pallas_call(
        _router_body,
        grid=(T // rb,),
        in_specs=[
            pl.BlockSpec((rb, DIM), lambda g: (g, 0)),
            pl.BlockSpec((E, DIM), lambda g: (0, 0)),
        ],
        out_specs=[
            pl.BlockSpec((rb, TOPK), lambda g: (g, 0)),
            pl.BlockSpec((rb, TOPK), lambda g: (g, 0)),
        ],
        out_shape=[
            jax.ShapeDtypeStruct((T, TOPK), jnp.float32),
            jax.ShapeDtypeStruct((T, TOPK), jnp.int32),
        ],
    )(xf, gate_w)


def _route_metadata(topi):
    flat_e = topi.reshape(-1)
    eids = jnp.arange(E, dtype=jnp.int32)
    oh = (flat_e[:, None] == eids[None, :]).astype(jnp.int32)
    csum = jnp.cumsum(oh, axis=0)
    rank = jnp.sum(csum * oh, axis=1) - 1
    counts = csum[-1]
    padded = ((counts + TB - 1) // TB) * TB
    ends = jnp.cumsum(padded)
    offs = ends - padded
    dest = jnp.sum(oh * offs[None, :], axis=1) + rank
    cb = ends // TB
    gids = jnp.arange(G, dtype=jnp.int32)
    block_expert = jnp.minimum(
        jnp.sum((gids[:, None] >= cb[None, :]).astype(jnp.int32), axis=1),
        E - 1).astype(jnp.int32)
    block_valid = (gids < cb[E - 1]).astype(jnp.int32)
    d2 = dest.reshape(T, TOPK)
    return d2[:, 0], d2[:, 1], block_expert, block_valid


def _sc_dispatch_body(x_hbm, d0_hbm, d1_hbm, out_hbm,
                      d0_v, d1_v, buf0, buf1, rs0, rs1, s00, s01, s10, s11):
    wid = lax.axis_index("s") * NC + lax.axis_index("c")
    base = wid * TPW
    pltpu.sync_copy(d0_hbm.at[wid], d0_v)
    pltpu.sync_copy(d1_hbm.at[wid], d1_v)
    bufs = (buf0, buf1)
    rsems = (rs0, rs1)
    ssems = ((s00, s01), (s10, s11))
    reads = [None, None]
    scats = [None, None, None, None]
    for c in range(DNCH):
        b = c % 2
        if scats[2 * b] is not None:
            scats[2 * b].wait()
            scats[2 * b + 1].wait()
        reads[b] = pltpu.async_copy(
            x_hbm.at[pl.ds(base + c * DCH, DCH)], bufs[b], rsems[b])
        reads[b].wait()
        scats[2 * b] = pltpu.async_copy(
            bufs[b], out_hbm.at[d0_v.at[c]], ssems[b][0])
        scats[2 * b + 1] = pltpu.async_copy(
            bufs[b], out_hbm.at[d1_v.at[c]], ssems[b][1])
    for cp in scats:
        if cp is not None:
            cp.wait()


def _sc_dispatch(xf, d0r, d1r):
    mesh = plsc.VectorSubcoreMesh(core_axis_name="c", subcore_axis_name="s")
    f = functools.partial(
        pl.kernel,
        mesh=mesh,
        out_type=jax.ShapeDtypeStruct((NROWS, DIM), jnp.float32),
        scratch_types=(
            [pltpu.VMEM((DNCH, DCH), jnp.int32)] * 2
            + [pltpu.VMEM((DCH, DIM), jnp.float32)] * 2
            + [pltpu.SemaphoreType.DMA] * 6
        ),
    )(_sc_dispatch_body)
    return f(xf, d0r, d1r)


def _gmm_body(be_ref, bv_ref, x_ref, w1_ref, w3_ref, w2_ref, out_ref):
    g = pl.program_id(0)

    @pl.when(bv_ref[g] != 0)
    def _():
        xb = x_ref[...]
        w1e = w1_ref[0]
        w3e = w3_ref[0]
        w2e = w2_ref[0]
        h1 = lax.dot_general(xb, w1e, (((1,), (1,)), ((), ())),
                             preferred_element_type=jnp.float32)
        h3 = lax.dot_general(xb, w3e, (((1,), (1,)), ((), ())),
                             preferred_element_type=jnp.float32)
        hh = h1 * (1.0 / (1.0 + jnp.exp(-h1))) * h3
        out_ref[...] = lax.dot_general(hh, w2e, (((1,), (1,)), ((), ())),
                                       preferred_element_type=jnp.float32)


def _gmm(block_expert, block_valid, xs, w1, w3, w2):
    grid_spec = pltpu.PrefetchScalarGridSpec(
        num_scalar_prefetch=2,
        grid=(G,),
        in_specs=[
            pl.BlockSpec((TB, DIM), lambda g, be, bv: (g, 0)),
            pl.BlockSpec((1, INTER, DIM), lambda g, be, bv: (be[g], 0, 0)),
            pl.BlockSpec((1, INTER, DIM), lambda g, be, bv: (be[g], 0, 0)),
            pl.BlockSpec((1, DIM, INTER), lambda g, be, bv: (be[g], 0, 0)),
        ],
        out_specs=pl.BlockSpec((TB, DIM), lambda g, be, bv: (g, 0)),
    )
    return pl.pallas_call(
        _gmm_body,
        grid_spec=grid_spec,
        out_shape=jax.ShapeDtypeStruct((NROWS, DIM), jnp.float32),
        compiler_params=pltpu.CompilerParams(
            dimension_semantics=("arbitrary",),
            vmem_limit_bytes=100 * 1024 * 1024),
    )(block_expert, block_valid, xs, w1, w3, w2)


def _shared_body(x_ref, sw1_ref, sw3_ref, sw2_ref, z_ref):
    xb = x_ref[...]
    h1 = lax.dot_general(xb, sw1_ref[...], (((1,), (1,)), ((), ())),
                         preferred_element_type=jnp.float32)
    h3 = lax.dot_general(xb, sw3_ref[...], (((1,), (1,)), ((), ())),
                         preferred_element_type=jnp.float32)
    hh = h1 * (1.0 / (1.0 + jnp.exp(-h1))) * h3
    z_ref[...] = lax.dot_general(hh, sw2_ref[...], (((1,), (1,)), ((), ())),
                                 preferred_element_type=jnp.float32)


def _shared(xf, sw1, sw3, sw2):
    sb = 256
    return pl.pallas_call(
        _shared_body,
        grid=(T // sb,),
        in_specs=[
            pl.BlockSpec((sb, DIM), lambda g: (g, 0)),
            pl.BlockSpec((INTER, DIM), lambda g: (0, 0)),
            pl.BlockSpec((INTER, DIM), lambda g: (0, 0)),
            pl.BlockSpec((DIM, INTER), lambda g: (0, 0)),
        ],
        out_specs=pl.BlockSpec((sb, DIM), lambda g: (g, 0)),
        out_shape=jax.ShapeDtypeStruct((T, DIM), jnp.float32),
        compiler_params=pltpu.CompilerParams(
            dimension_semantics=("arbitrary",),
            vmem_limit_bytes=100 * 1024 * 1024),
    )(xf, sw1, sw3, sw2)


def _sc_combine_body(contrib_hbm, d0_hbm, d1_hbm, z_hbm, w0b_hbm, w1b_hbm,
                     out_hbm,
                     d0_v, d1_v, w0_v, w1_v,
                     r0a, r1a, za, r0b, r1b, zb,
                     sa0, sa1, sa2, sb0, sb1, sb2, wsa, wsb):
    wid = lax.axis_index("s") * NC + lax.axis_index("c")
    base = wid * TPW
    pltpu.sync_copy(d0_hbm.at[pl.ds(base, TPW)], d0_v)
    pltpu.sync_copy(d1_hbm.at[pl.ds(base, TPW)], d1_v)
    pltpu.sync_copy(w0b_hbm.at[pl.ds(base, TPW)], w0_v)
    pltpu.sync_copy(w1b_hbm.at[pl.ds(base, TPW)], w1_v)

    def accum(c, dst, s0, s1):
        wv0 = [w0_v[c * CCH + i, :] for i in range(CCH)]
        wv1 = [w1_v[c * CCH + i, :] for i in range(CCH)]

        def add_body(j, carry2):
            sl = pl.ds(j * L, L)
            for i in range(CCH):
                dst[i, sl] = dst[i, sl] + wv0[i] * s0[i, sl] + wv1[i] * s1[i, sl]
            return carry2
        lax.fori_loop(0, DIM // L, add_body, 0)

    def pair(i, carry):
        c0 = i * 2
        c1 = c0 + 1
        ta = base + c0 * CCH
        tb = base + c1 * CCH
        ga0 = pltpu.async_copy(contrib_hbm.at[d0_v.at[pl.ds(c0 * CCH, CCH)]],
                               r0a, sa0)
        ga1 = pltpu.async_copy(contrib_hbm.at[d1_v.at[pl.ds(c0 * CCH, CCH)]],
                               r1a, sa1)
        ga2 = pltpu.async_copy(z_hbm.at[pl.ds(ta, CCH)], za, sa2)
        gb0 = pltpu.async_copy(contrib_hbm.at[d0_v.at[pl.ds(c1 * CCH, CCH)]],
                               r0b, sb0)
        gb1 = pltpu.async_copy(contrib_hbm.at[d1_v.at[pl.ds(c1 * CCH, CCH)]],
                               r1b, sb1)
        gb2 = pltpu.async_copy(z_hbm.at[pl.ds(tb, CCH)], zb, sb2)
        ga0.wait()
        ga1.wait()
        ga2.wait()
        accum(c0, za, r0a, r1a)
        wa = pltpu.async_copy(za, out_hbm.at[pl.ds(ta, CCH)], wsa)
        gb0.wait()
        gb1.wait()
        gb2.wait()
        accum(c1, zb, r0b, r1b)
        wb = pltpu.async_copy(zb, out_hbm.at[pl.ds(tb, CCH)], wsb)
        wa.wait()
        wb.wait()
        return carry

    lax.fori_loop(0, TPW // (2 * CCH), pair, 0)


def _sc_combine(contrib, d0, d1, z, w0b, w1b):
    mesh = plsc.VectorSubcoreMesh(core_axis_name="c", subcore_axis_name="s")
    f = functools.partial(
        pl.kernel,
        mesh=mesh,
        out_type=jax.ShapeDtypeStruct((T, DIM), jnp.float32),
        scratch_types=(
            [pltpu.VMEM((TPW,), jnp.int32)] * 2
            + [pltpu.VMEM((TPW, L), jnp.float32)] * 2
            + [pltpu.VMEM((CCH, DIM), jnp.float32)] * 6
            + [pltpu.SemaphoreType.DMA] * 8
        ),
    )(_sc_combine_body)
    return f(contrib, d0, d1, z, w0b, w1b)


def kernel(x, gate_w, w1, w2, w3, sw1, sw2, sw3):
    shape = x.shape
    xf = x.reshape(T, DIM)
    topv, topi = _router(xf, gate_w)
    d0, d1, block_expert, block_valid = _route_metadata(topi)
    d0r = d0.reshape(NW, DNCH, DCH)
    d1r = d1.reshape(NW, DNCH, DCH)
    w0b = jnp.broadcast_to(topv[:, 0:1], (T, L))
    w1b = jnp.broadcast_to(topv[:, 1:2], (T, L))
    xs = _sc_dispatch(xf, d0r, d1r)
    z = _shared(xf, sw1, sw3, sw2)
    contrib = _gmm(block_expert, block_valid, xs, w1, w3, w2)
    y = _sc_combine(contrib, d0, d1, z, w0b, w1b)
    return y.reshape(shape)


# speedup vs baseline: 1.2751x; 1.0226x over previous
import functools

import jax
import jax.numpy as jnp
from jax import lax
from jax.experimental import pallas as pl
from jax.experimental.pallas import tpu as pltpu
from jax.experimental.pallas import tpu_sc as plsc

E = 16
TOPK = 2
DIM = 2048
INTER = 1024
T = 2048
TB = 384
G = (T * TOPK) // TB + E
NROWS = G * TB

NC = 2
NS = 16
L = 16
NW = NC * NS

TPW = T // NW
DCH = 16
DNCH = TPW // DCH
CCH = 8


def _router_body(x_ref, gw_ref, topi_ref, w0_ref, w1_ref):
    xb = x_ref[...]
    gw = gw_ref[...]
    logits = lax.dot_general(xb, gw, (((1,), (1,)), ((), ())),
                             preferred_element_type=jnp.float32)
    m = jnp.max(logits, axis=1, keepdims=True)
    ex = jnp.exp(logits - m)
    s = ex / jnp.sum(ex, axis=1, keepdims=True)
    col = lax.broadcasted_iota(jnp.int32, s.shape, 1)
    v1 = jnp.max(s, axis=1, keepdims=True)
    i1 = jnp.min(jnp.where(s == v1, col, E), axis=1, keepdims=True)
    s2 = jnp.where(col == i1, jnp.float32(-jnp.inf), s)
    v2 = jnp.max(s2, axis=1, keepdims=True)
    i2 = jnp.min(jnp.where(s2 == v2, col, E), axis=1, keepdims=True)
    topi_ref[...] = jnp.concatenate([i1, i2], axis=1)
    w0_ref[...] = jnp.broadcast_to(v1, v1.shape[:1] + (L,))
    w1_ref[...] = jnp.broadcast_to(v2, v2.shape[:1] + (L,))


def _router(xf, gate_w):
    rb = 256
    return pl.pallas_call(
        _router_body,
        grid=(T // rb,),
        in_specs=[
            pl.BlockSpec((rb, DIM), lambda g: (g, 0)),
            pl.BlockSpec((E, DIM), lambda g: (0, 0)),
        ],
        out_specs=[
            pl.BlockSpec((rb, TOPK), lambda g: (g, 0)),
            pl.BlockSpec((rb, L), lambda g: (g, 0)),
            pl.BlockSpec((rb, L), lambda g: (g, 0)),
        ],
        out_shape=[
            jax.ShapeDtypeStruct((T, TOPK), jnp.int32),
            jax.ShapeDtypeStruct((T, L), jnp.float32),
            jax.ShapeDtypeStruct((T, L), jnp.float32),
        ],
    )(xf, gate_w)


def _route_metadata(topi):
    flat_e = topi.reshape(-1)
    eids = jnp.arange(E, dtype=jnp.int32)
    oh = (flat_e[:, None] == eids[None, :]).astype(jnp.int32)
    csum = jnp.cumsum(oh, axis=0)
    rank = jnp.sum(csum * oh, axis=1) - 1
    counts = csum[-1]
    padded = ((counts + TB - 1) // TB) * TB
    ends = jnp.cumsum(padded)
    offs = ends - padded
    dest = jnp.sum(oh * offs[None, :], axis=1) + rank
    cb = ends // TB
    gids = jnp.arange(G, dtype=jnp.int32)
    block_expert = jnp.minimum(
        jnp.sum((gids[:, None] >= cb[None, :]).astype(jnp.int32), axis=1),
        E - 1).astype(jnp.int32)
    block_valid = (gids < cb[E - 1]).astype(jnp.int32)
    d2 = dest.reshape(T, TOPK)
    return d2[:, 0], d2[:, 1], block_expert, block_valid


def _sc_dispatch_body(x_hbm, d0_hbm, d1_hbm, out_hbm,
                      d0_v, d1_v, buf0, buf1, rs0, rs1, s00, s01, s10, s11):
    wid = lax.axis_index("s") * NC + lax.axis_index("c")
    base = wid * TPW
    pltpu.sync_copy(d0_hbm.at[wid], d0_v)
    pltpu.sync_copy(d1_hbm.at[wid], d1_v)
    bufs = (buf0, buf1)
    rsems = (rs0, rs1)
    ssems = ((s00, s01), (s10, s11))
    reads = [None, None]
    scats = [None, None, None, None]
    for c in range(DNCH):
        b = c % 2
        if scats[2 * b] is not None:
            scats[2 * b].wait()
            scats[2 * b + 1].wait()
        reads[b] = pltpu.async_copy(
            x_hbm.at[pl.ds(base + c * DCH, DCH)], bufs[b], rsems[b])
        reads[b].wait()
        scats[2 * b] = pltpu.async_copy(
            bufs[b], out_hbm.at[d0_v.at[c]], ssems[b][0])
        scats[2 * b + 1] = pltpu.async_copy(
            bufs[b], out_hbm.at[d1_v.at[c]], ssems[b][1])
    for cp in scats:
        if cp is not None:
            cp.wait()


def _sc_dispatch(xf, d0r, d1r):
    mesh = plsc.VectorSubcoreMesh(core_axis_name="c", subcore_axis_name="s")
    f = functools.partial(
        pl.kernel,
        mesh=mesh,
        out_type=jax.ShapeDtypeStruct((NROWS, DIM), jnp.float32),
        scratch_types=(
            [pltpu.VMEM((DNCH, DCH), jnp.int32)] * 2
            + [pltpu.VMEM((DCH, DIM), jnp.float32)] * 2
            + [pltpu.SemaphoreType.DMA] * 6
        ),
    )(_sc_dispatch_body)
    return f(xf, d0r, d1r)


def _gmm_body(be_ref, bv_ref, x_ref, w1_ref, w3_ref, w2_ref, out_ref):
    g = pl.program_id(0)

    @pl.when(bv_ref[g] != 0)
    def _():
        xb = x_ref[...]
        w1e = w1_ref[0]
        w3e = w3_ref[0]
        w2e = w2_ref[0]
        h1 = lax.dot_general(xb, w1e, (((1,), (1,)), ((), ())),
                             preferred_element_type=jnp.float32)
        h3 = lax.dot_general(xb, w3e, (((1,), (1,)), ((), ())),
                             preferred_element_type=jnp.float32)
        hh = h1 * (1.0 / (1.0 + jnp.exp(-h1))) * h3
        out_ref[...] = lax.dot_general(hh, w2e, (((1,), (1,)), ((), ())),
                                       preferred_element_type=jnp.float32)


def _gmm(block_expert, block_valid, xs, w1, w3, w2):
    grid_spec = pltpu.PrefetchScalarGridSpec(
        num_scalar_prefetch=2,
        grid=(G,),
        in_specs=[
            pl.BlockSpec((TB, DIM), lambda g, be, bv: (g, 0)),
            pl.BlockSpec((1, INTER, DIM), lambda g, be, bv: (be[g], 0, 0)),
            pl.BlockSpec((1, INTER, DIM), lambda g, be, bv: (be[g], 0, 0)),
            pl.BlockSpec((1, DIM, INTER), lambda g, be, bv: (be[g], 0, 0)),
        ],
        out_specs=pl.BlockSpec((TB, DIM), lambda g, be, bv: (g, 0)),
    )
    return pl.pallas_call(
        _gmm_body,
        grid_spec=grid_spec,
        out_shape=jax.ShapeDtypeStruct((NROWS, DIM), jnp.float32),
        compiler_params=pltpu.CompilerParams(
            dimension_semantics=("arbitrary",),
            vmem_limit_bytes=100 * 1024 * 1024),
    )(block_expert, block_valid, xs, w1, w3, w2)


def _shared_body(x_ref, sw1_ref, sw3_ref, sw2_ref, z_ref):
    xb = x_ref[...]
    h1 = lax.dot_general(xb, sw1_ref[...], (((1,), (1,)), ((), ())),
                         preferred_element_type=jnp.float32)
    h3 = lax.dot_general(xb, sw3_ref[...], (((1,), (1,)), ((), ())),
                         preferred_element_type=jnp.float32)
    hh = h1 * (1.0 / (1.0 + jnp.exp(-h1))) * h3
    z_ref[...] = lax.dot_general(hh, sw2_ref[...], (((1,), (1,)), ((), ())),
                                 preferred_element_type=jnp.float32)


def _shared(xf, sw1, sw3, sw2):
    sb = 512
    return pl.pallas_call(
        _shared_body,
        grid=(T // sb,),
        in_specs=[
            pl.BlockSpec((sb, DIM), lambda g: (g, 0)),
            pl.BlockSpec((INTER, DIM), lambda g: (0, 0)),
            pl.BlockSpec((INTER, DIM), lambda g: (0, 0)),
            pl.BlockSpec((DIM, INTER), lambda g: (0, 0)),
        ],
        out_specs=pl.BlockSpec((sb, DIM), lambda g: (g, 0)),
        out_shape=jax.ShapeDtypeStruct((T, DIM), jnp.float32),
        compiler_params=pltpu.CompilerParams(
            dimension_semantics=("arbitrary",),
            vmem_limit_bytes=100 * 1024 * 1024),
    )(xf, sw1, sw3, sw2)


def _sc_combine_body(contrib_hbm, d0_hbm, d1_hbm, z_hbm, w0b_hbm, w1b_hbm,
                     out_hbm,
                     d0_v, d1_v, w0_v, w1_v,
                     r0a, r1a, za, r0b, r1b, zb,
                     sa0, sa1, sa2, sb0, sb1, sb2, wsa, wsb):
    wid = lax.axis_index("s") * NC + lax.axis_index("c")
    base = wid * TPW
    pltpu.sync_copy(d0_hbm.at[pl.ds(base, TPW)], d0_v)
    pltpu.sync_copy(d1_hbm.at[pl.ds(base, TPW)], d1_v)
    pltpu.sync_copy(w0b_hbm.at[pl.ds(base, TPW)], w0_v)
    pltpu.sync_copy(w1b_hbm.at[pl.ds(base, TPW)], w1_v)

    def accum(c, dst, s0, s1):
        wv0 = [w0_v[c * CCH + i, :] for i in range(CCH)]
        wv1 = [w1_v[c * CCH + i, :] for i in range(CCH)]

        def add_body(j, carry2):
            sl = pl.ds(j * L, L)
            for i in range(CCH):
                dst[i, sl] = dst[i, sl] + wv0[i] * s0[i, sl] + wv1[i] * s1[i, sl]
            return carry2
        lax.fori_loop(0, DIM // L, add_body, 0)

    def pair(i, carry):
        c0 = i * 2
        c1 = c0 + 1
        ta = base + c0 * CCH
        tb = base + c1 * CCH
        ga0 = pltpu.async_copy(contrib_hbm.at[d0_v.at[pl.ds(c0 * CCH, CCH)]],
                               r0a, sa0)
        ga1 = pltpu.async_copy(contrib_hbm.at[d1_v.at[pl.ds(c0 * CCH, CCH)]],
                               r1a, sa1)
        ga2 = pltpu.async_copy(z_hbm.at[pl.ds(ta, CCH)], za, sa2)
        gb0 = pltpu.async_copy(contrib_hbm.at[d0_v.at[pl.ds(c1 * CCH, CCH)]],
                               r0b, sb0)
        gb1 = pltpu.async_copy(contrib_hbm.at[d1_v.at[pl.ds(c1 * CCH, CCH)]],
                               r1b, sb1)
        gb2 = pltpu.async_copy(z_hbm.at[pl.ds(tb, CCH)], zb, sb2)
        ga0.wait()
        ga1.wait()
        ga2.wait()
        accum(c0, za, r0a, r1a)
        wa = pltpu.async_copy(za, out_hbm.at[pl.ds(ta, CCH)], wsa)
        gb0.wait()
        gb1.wait()
        gb2.wait()
        accum(c1, zb, r0b, r1b)
        wb = pltpu.async_copy(zb, out_hbm.at[pl.ds(tb, CCH)], wsb)
        wa.wait()
        wb.wait()
        return carry

    lax.fori_loop(0, TPW // (2 * CCH), pair, 0)


def _sc_combine(contrib, d0, d1, z, w0b, w1b):
    mesh = plsc.VectorSubcoreMesh(core_axis_name="c", subcore_axis_name="s")
    f = functools.partial(
        pl.kernel,
        mesh=mesh,
        out_type=jax.ShapeDtypeStruct((T, DIM), jnp.float32),
        scratch_types=(
            [pltpu.VMEM((TPW,), jnp.int32)] * 2
            + [pltpu.VMEM((TPW, L), jnp.float32)] * 2
            + [pltpu.VMEM((CCH, DIM), jnp.float32)] * 6
            + [pltpu.SemaphoreType.DMA] * 8
        ),
    )(_sc_combine_body)
    return f(contrib, d0, d1, z, w0b, w1b)


def kernel(x, gate_w, w1, w2, w3, sw1, sw2, sw3):
    shape = x.shape
    xf = x.reshape(T, DIM)
    topi, w0b, w1b = _router(xf, gate_w)
    d0, d1, block_expert, block_valid = _route_metadata(topi)
    d0r = d0.reshape(NW, DNCH, DCH)
    d1r = d1.reshape(NW, DNCH, DCH)
    xs = _sc_dispatch(xf, d0r, d1r)
    z = _shared(xf, sw1, sw3, sw2)
    contrib = _gmm(block_expert, block_valid, xs, w1, w3, w2)
    y = _sc_combine(contrib, d0, d1, z, w0b, w1b)
    return y.reshape(shape)
